# baseline (speedup 1.0000x reference)
import jax
import jax.numpy as jnp
from jax import lax
from jax.experimental import pallas as pl
from jax.experimental.pallas import tpu as pltpu
from jax.experimental.pallas import tpu_sc as plsc

B = 16384
D = 64
NC = 2
NS = 16
NW = NC * NS
BPW = B // NW
CHUNK = 128
NCHUNK = BPW // CHUNK


def _project_kernel(cat_ref, w_ref, out_ref):
    out_ref[...] = jnp.dot(
        cat_ref[...], w_ref[...].T, preferred_element_type=jnp.float32
    )


def _project(category_x, cat2item_w):
    n_cat = category_x.shape[0]
    return pl.pallas_call(
        _project_kernel,
        out_shape=jax.ShapeDtypeStruct((n_cat, D), jnp.float32),
    )(category_x, cat2item_w)


def _sc_body(uidx_hbm, iidx_hbm, cidx_hbm, user_w, user_b16, item_x, proj,
             out_hbm, uidx_v, iidx_v, cidx_v, bidx_v, col_v,
             w_v, x_v, p_v, b_v, out_v, sem):
    wid = lax.axis_index("s") * NC + lax.axis_index("c")

    pltpu.sync_copy(uidx_hbm.at[wid], uidx_v)
    pltpu.sync_copy(iidx_hbm.at[wid], iidx_v)
    pltpu.sync_copy(cidx_hbm.at[wid], cidx_v)

    for j in range(NCHUNK):
        for o in range(CHUNK // 16):
            s = pl.ds(o * 16, 16)
            idx = uidx_v[j, s]
            bidx_v[j, s] = lax.shift_right_logical(idx, 4)
            col_v[pl.ds(j * CHUNK + o * 16, 16)] = idx & 15

    copies = []
    for j in range(NCHUNK):
        rows = pl.ds(j * CHUNK, CHUNK)
        copies.append(pltpu.async_copy(user_w.at[uidx_v.at[j]], w_v.at[rows], sem))
        copies.append(pltpu.async_copy(item_x.at[iidx_v.at[j]], x_v.at[rows], sem))
        copies.append(pltpu.async_copy(proj.at[cidx_v.at[j]], p_v.at[rows], sem))
        copies.append(pltpu.async_copy(user_b16.at[bidx_v.at[j]], b_v.at[rows], sem))
    for c in copies:
        c.wait()

    lane = lax.iota(jnp.int32, 16)

    def group(g, _):
        res = jnp.zeros((16,), jnp.float32)
        base = g * 16
        for r in range(16):
            i = base + r
            acc = w_v[i, pl.ds(0, 16)] * (x_v[i, pl.ds(0, 16)] + p_v[i, pl.ds(0, 16)])
            for k in range(1, D // 16):
                s = pl.ds(k * 16, 16)
                acc = acc + w_v[i, s] * (x_v[i, s] + p_v[i, s])
            res = jnp.where(lane == r, jnp.sum(acc), res)
        bias = plsc.load_gather(b_v, [base + lane, col_v[pl.ds(base, 16)]])
        out_v[pl.ds(base, 16)] = res + bias
        return 0

    lax.fori_loop(0, BPW // 16, group, 0)

    pltpu.sync_copy(out_v, out_hbm.at[pl.ds(wid * BPW, BPW)])


@jax.jit
def _sc_gather_dot(uidx3, iidx3, cidx3, user_w, user_b16, item_x, proj):
    mesh = plsc.VectorSubcoreMesh(core_axis_name="c", subcore_axis_name="s")
    return pl.kernel(
        _sc_body,
        out_type=jax.ShapeDtypeStruct((B,), jnp.float32),
        mesh=mesh,
        compiler_params=pltpu.CompilerParams(
            needs_layout_passes=False, use_tc_tiling_on_sc=False
        ),
        scratch_types=[
            pltpu.VMEM((NCHUNK, CHUNK), jnp.int32),
            pltpu.VMEM((NCHUNK, CHUNK), jnp.int32),
            pltpu.VMEM((NCHUNK, CHUNK), jnp.int32),
            pltpu.VMEM((NCHUNK, CHUNK), jnp.int32),
            pltpu.VMEM((BPW,), jnp.int32),
            pltpu.VMEM((BPW, D), jnp.float32),
            pltpu.VMEM((BPW, D), jnp.float32),
            pltpu.VMEM((BPW, D), jnp.float32),
            pltpu.VMEM((BPW, 16), jnp.float32),
            pltpu.VMEM((BPW,), jnp.float32),
            pltpu.SemaphoreType.DMA,
        ],
    )(uidx3, iidx3, cidx3, user_w, user_b16, item_x, proj)


def kernel(user_idx, item_idx, category_idx, user_w, user_b, item_x,
           category_x, cat2item_w):
    proj = _project(category_x, cat2item_w)
    uidx3 = user_idx.astype(jnp.int32).reshape(NW, NCHUNK, CHUNK)
    iidx3 = item_idx.astype(jnp.int32).reshape(NW, NCHUNK, CHUNK)
    cidx3 = category_idx.astype(jnp.int32).reshape(NW, NCHUNK, CHUNK)
    user_b16 = user_b.reshape(user_b.shape[0] // 16, 16)
    return _sc_gather_dot(uidx3, iidx3, cidx3, user_w, user_b16, item_x, proj)


# speedup vs baseline: 1.1914x; 1.1914x over previous
import jax
import jax.numpy as jnp
from jax import lax
from jax.experimental import pallas as pl
from jax.experimental.pallas import tpu as pltpu
from jax.experimental.pallas import tpu_sc as plsc

B = 16384
D = 64
NC = 2
NS = 16
NW = NC * NS
BPW = B // NW
CH = 128
NCH = BPW // CH


def _project_kernel(cat_ref, w_ref, out_ref):
    out_ref[...] = jnp.dot(
        cat_ref[...], w_ref[...].T, preferred_element_type=jnp.float32
    )


def _project(category_x, cat2item_w):
    n_cat = category_x.shape[0]
    return pl.pallas_call(
        _project_kernel,
        out_shape=jax.ShapeDtypeStruct((n_cat, D), jnp.float32),
    )(category_x, cat2item_w)


def _sc_body(uidx_hbm, iidx_hbm, cidx_hbm, uw_hbm, ub_hbm, ix_hbm, pj_hbm,
             out_hbm, uidx_v, iidx_v, cidx_v, w_v, x_v, p_v, b_v, out_v, sem):
    wid = lax.axis_index("s") * NC + lax.axis_index("c")
    pltpu.sync_copy(uidx_hbm.at[wid], uidx_v)
    pltpu.sync_copy(iidx_hbm.at[wid], iidx_v)
    pltpu.sync_copy(cidx_hbm.at[wid], cidx_v)
    lane = lax.iota(jnp.int32, 16)
    zeros16 = jnp.zeros((16,), jnp.int32)

    def chunk(c, _):
        base = c * CH

        def fire(g, _):
            gb = g * 16
            uvec = uidx_v[pl.ds(base + gb, 16)]
            ivec = iidx_v[pl.ds(base + gb, 16)]
            cvec = cidx_v[pl.ds(base + gb, 16)]
            for r in range(16):
                row = pl.ds(gb + r, 1)
                pltpu.make_async_copy(
                    uw_hbm.at[pl.ds(uvec[r], 1)], w_v.at[row], sem
                ).start()
                pltpu.make_async_copy(
                    ub_hbm.at[pl.ds(uvec[r], 1)], b_v.at[row], sem
                ).start()
                pltpu.make_async_copy(
                    ix_hbm.at[pl.ds(ivec[r], 1)], x_v.at[row], sem
                ).start()
                pltpu.make_async_copy(
                    pj_hbm.at[pl.ds(cvec[r], 1)], p_v.at[row], sem
                ).start()
            return 0

        lax.fori_loop(0, CH // 16, fire, 0)

        def drain(i, _):
            row = pl.ds(i, 1)
            pltpu.make_async_copy(uw_hbm.at[pl.ds(0, 1)], w_v.at[row], sem).wait()
            pltpu.make_async_copy(ub_hbm.at[pl.ds(0, 1)], b_v.at[row], sem).wait()
            pltpu.make_async_copy(ix_hbm.at[pl.ds(0, 1)], x_v.at[row], sem).wait()
            pltpu.make_async_copy(pj_hbm.at[pl.ds(0, 1)], p_v.at[row], sem).wait()
            return 0

        lax.fori_loop(0, CH, drain, 0)

        def group(g, _):
            res = jnp.zeros((16,), jnp.float32)
            gb = g * 16
            for r in range(16):
                i = gb + r
                acc = w_v[i, pl.ds(0, 16)] * (x_v[i, pl.ds(0, 16)] + p_v[i, pl.ds(0, 16)])
                for k in range(1, D // 16):
                    s = pl.ds(k * 16, 16)
                    acc = acc + w_v[i, s] * (x_v[i, s] + p_v[i, s])
                res = jnp.where(lane == r, jnp.sum(acc), res)
            bias = plsc.load_gather(b_v, [gb + lane, zeros16])
            out_v[pl.ds(base + gb, 16)] = res + bias
            return 0

        lax.fori_loop(0, CH // 16, group, 0)
        return 0

    lax.fori_loop(0, NCH, chunk, 0)

    pltpu.sync_copy(out_v, out_hbm.at[pl.ds(wid * BPW, BPW)])


@jax.jit
def _sc_gather_dot(uidx2, iidx2, cidx2, user_w, user_b, item_x, proj):
    mesh = plsc.VectorSubcoreMesh(core_axis_name="c", subcore_axis_name="s")
    return pl.kernel(
        _sc_body,
        out_type=jax.ShapeDtypeStruct((B,), jnp.float32),
        mesh=mesh,
        compiler_params=pltpu.CompilerParams(needs_layout_passes=False),
        scratch_types=[
            pltpu.VMEM((BPW,), jnp.int32),
            pltpu.VMEM((BPW,), jnp.int32),
            pltpu.VMEM((BPW,), jnp.int32),
            pltpu.VMEM((CH, D), jnp.float32),
            pltpu.VMEM((CH, D), jnp.float32),
            pltpu.VMEM((CH, D), jnp.float32),
            pltpu.VMEM((CH, 1), jnp.float32),
            pltpu.VMEM((BPW,), jnp.float32),
            pltpu.SemaphoreType.DMA,
        ],
    )(uidx2, iidx2, cidx2, user_w, user_b, item_x, proj)


def kernel(user_idx, item_idx, category_idx, user_w, user_b, item_x,
           category_x, cat2item_w):
    proj = _project(category_x, cat2item_w)
    uidx2 = user_idx.astype(jnp.int32).reshape(NW, BPW)
    iidx2 = item_idx.astype(jnp.int32).reshape(NW, BPW)
    cidx2 = category_idx.astype(jnp.int32).reshape(NW, BPW)
    return _sc_gather_dot(uidx2, iidx2, cidx2, user_w, user_b, item_x, proj)
